# baseline (device time: 391366 ns/iter reference)
import contextlib
import os

import jax
import jax.numpy as jnp
from jax import lax
from jax.experimental import pallas as pl
from jax.experimental.pallas import tpu as pltpu

if os.environ.get("PROF_SCOPES"):
    def _scope(n):
        return jax.named_scope(n)
else:
    def _scope(n):
        return contextlib.nullcontext()

N_DEV = 8
M = 4096
K_SHARD = 512
N_OUT = 2048
HALF = N_OUT // 2
CHUNK = M // N_DEV
SUB = CHUNK // 2

N_STEPS = 2 * (N_DEV - 1)
SLOTS = 4


def _silu(v):
    return v * (1.0 / (1.0 + jnp.exp(-v)))


def _body(x_ref, w_ref, dummy_hbm, out_hbm, comm_cw, comm_ccw,
          send_cw, recv_cw, send_ccw, recv_ccw,
          out_sem_a, out_sem_b, credit_cw, credit_ccw):
    my = lax.axis_index("i")
    left = lax.rem(my + N_DEV - 1, N_DEV)
    right = lax.rem(my + 1, N_DEV)

    def pchunk(c, lo, hi):
        return jnp.dot(
            x_ref[pl.ds(c * CHUNK, CHUNK), :], w_ref[:, lo:hi],
            preferred_element_type=jnp.float32,
        )

    with _scope("p_seed"):
        comm_cw[0] = pchunk(my, 0, HALF)
        comm_ccw[0] = pchunk(my, HALF, N_OUT)

    barrier_sem = pltpu.get_barrier_semaphore()
    for nbr in (left, right):
        pl.semaphore_signal(
            barrier_sem, inc=1,
            device_id=(nbr,), device_id_type=pl.DeviceIdType.MESH,
        )
    pl.semaphore_wait(barrier_sem, 2)

    pending_stores = []
    for t in range(N_STEPS):
        s_slot = t % SLOTS
        r_slot = (t + 1) % SLOTS
        if t >= SLOTS - 1:
            with _scope("p_credit"):
                pl.semaphore_wait(credit_cw, 1)
                pl.semaphore_wait(credit_ccw, 1)
        rdmas = {}
        with _scope("p_issue"):
            for ring, comm, ssem, rsem, dev in (
                ("cw", comm_cw, send_cw, recv_cw, right),
                ("ccw", comm_ccw, send_ccw, recv_ccw, left),
            ):
                for sub in range(2):
                    r = pltpu.make_async_remote_copy(
                        src_ref=comm.at[s_slot, pl.ds(sub * SUB, SUB), :],
                        dst_ref=comm.at[r_slot, pl.ds(sub * SUB, SUB), :],
                        send_sem=ssem.at[s_slot, sub],
                        recv_sem=rsem.at[r_slot, sub],
                        device_id=(dev,), device_id_type=pl.DeviceIdType.MESH,
                    )
                    r.start()
                    rdmas[(ring, sub)] = r
        if t < N_DEV - 1:
            with _scope("p_mxu"):
                c_cw = lax.rem(my - (t + 1) + N_DEV, N_DEV)
                c_ccw = lax.rem(my + t + 1, N_DEV)
                pa = pchunk(c_cw, 0, HALF)
                pb = pchunk(c_ccw, HALF, N_OUT)

        with _scope("p_waitacc"):
            for sub in range(2):
                lo, hi = sub * SUB, (sub + 1) * SUB
                for ring, comm, p in (("cw", comm_cw, "pa"), ("ccw", comm_ccw, "pb")):
                    rdmas[(ring, sub)].wait()
                    v = pa if p == "pa" else pb
                    if t < N_DEV - 2:
                        comm[r_slot, lo:hi, :] = comm[r_slot, lo:hi, :] + v[lo:hi, :]
                    elif t == N_DEV - 2:
                        comm[r_slot, lo:hi, :] = _silu(
                            comm[r_slot, lo:hi, :] + v[lo:hi, :])

        if pending_stores:
            with _scope("p_storewait"):
                for st in pending_stores:
                    st.wait()
            pending_stores = []
        if t < N_STEPS - (SLOTS - 1):
            pl.semaphore_signal(
                credit_cw, inc=1,
                device_id=(left,), device_id_type=pl.DeviceIdType.MESH,
            )
            pl.semaphore_signal(
                credit_ccw, inc=1,
                device_id=(right,), device_id_type=pl.DeviceIdType.MESH,
            )
        if t >= N_DEV - 2:
            c_a = lax.rem(my - (t - (N_DEV - 1)) + N_DEV, N_DEV)
            c_b = lax.rem(my + (t - (N_DEV - 1)) + N_DEV, N_DEV)
            st_a = pltpu.make_async_copy(
                comm_cw.at[r_slot],
                out_hbm.at[pl.ds(c_a * CHUNK, CHUNK), pl.ds(0, HALF)],
                out_sem_a)
            st_b = pltpu.make_async_copy(
                comm_ccw.at[r_slot],
                out_hbm.at[pl.ds(c_b * CHUNK, CHUNK), pl.ds(HALF, HALF)],
                out_sem_b)
            st_a.start()
            st_b.start()
            pending_stores = [st_a, st_b]

    for st in pending_stores:
        st.wait()


def kernel(x, w_mat):
    dummy = jnp.zeros((M, N_OUT), jnp.float32)
    return pl.pallas_call(
        _body,
        out_shape=jax.ShapeDtypeStruct((M, N_OUT), jnp.float32),
        in_specs=[
            pl.BlockSpec(memory_space=pltpu.VMEM),
            pl.BlockSpec(memory_space=pltpu.VMEM),
            pl.BlockSpec(memory_space=pltpu.MemorySpace.HBM),
        ],
        out_specs=pl.BlockSpec(memory_space=pltpu.MemorySpace.HBM),
        input_output_aliases={2: 0},
        scratch_shapes=[
            pltpu.VMEM((SLOTS, CHUNK, HALF), jnp.float32),
            pltpu.VMEM((SLOTS, CHUNK, HALF), jnp.float32),
            pltpu.SemaphoreType.DMA((SLOTS, 2)),
            pltpu.SemaphoreType.DMA((SLOTS, 2)),
            pltpu.SemaphoreType.DMA((SLOTS, 2)),
            pltpu.SemaphoreType.DMA((SLOTS, 2)),
            pltpu.SemaphoreType.DMA,
            pltpu.SemaphoreType.DMA,
            pltpu.SemaphoreType.REGULAR,
            pltpu.SemaphoreType.REGULAR,
        ],
        compiler_params=pltpu.CompilerParams(collective_id=0),
    )(x, w_mat, dummy)


# device time: 382765 ns/iter; 1.0225x vs baseline; 1.0225x over previous
import contextlib
import os

import jax
import jax.numpy as jnp
from jax import lax
from jax.experimental import pallas as pl
from jax.experimental.pallas import tpu as pltpu

if os.environ.get("PROF_SCOPES"):
    def _scope(n):
        return jax.named_scope(n)
else:
    def _scope(n):
        return contextlib.nullcontext()

N_DEV = 8
M = 4096
K_SHARD = 512
N_OUT = 2048
HALF = N_OUT // 2
CHUNK = M // N_DEV
SUB = CHUNK // 2

N_STEPS = 2 * (N_DEV - 1)
SLOTS = 4


def _silu(v):
    return v * (1.0 / (1.0 + jnp.exp(-v)))


def _body(x_ref, w_ref, out_hbm, comm_cw, comm_ccw,
          send_cw, recv_cw, send_ccw, recv_ccw,
          out_sem_a, out_sem_b, credit_cw, credit_ccw):
    my = lax.axis_index("i")
    left = lax.rem(my + N_DEV - 1, N_DEV)
    right = lax.rem(my + 1, N_DEV)

    def pchunk(c, lo, hi):
        return jnp.dot(
            x_ref[pl.ds(c * CHUNK, CHUNK), :], w_ref[:, lo:hi],
            preferred_element_type=jnp.float32,
        )

    with _scope("p_seed"):
        comm_cw[0] = pchunk(my, 0, HALF)
        comm_ccw[0] = pchunk(my, HALF, N_OUT)

    barrier_sem = pltpu.get_barrier_semaphore()
    for nbr in (left, right):
        pl.semaphore_signal(
            barrier_sem, inc=1,
            device_id=(nbr,), device_id_type=pl.DeviceIdType.MESH,
        )
    pl.semaphore_wait(barrier_sem, 2)

    pending_stores = []
    for t in range(N_STEPS):
        s_slot = t % SLOTS
        r_slot = (t + 1) % SLOTS
        if t >= SLOTS - 1:
            with _scope("p_credit"):
                pl.semaphore_wait(credit_cw, 1)
                pl.semaphore_wait(credit_ccw, 1)
        rdmas = {}
        with _scope("p_issue"):
            for ring, comm, ssem, rsem, dev in (
                ("cw", comm_cw, send_cw, recv_cw, right),
                ("ccw", comm_ccw, send_ccw, recv_ccw, left),
            ):
                for sub in range(2):
                    r = pltpu.make_async_remote_copy(
                        src_ref=comm.at[s_slot, pl.ds(sub * SUB, SUB), :],
                        dst_ref=comm.at[r_slot, pl.ds(sub * SUB, SUB), :],
                        send_sem=ssem.at[s_slot, sub],
                        recv_sem=rsem.at[r_slot, sub],
                        device_id=(dev,), device_id_type=pl.DeviceIdType.MESH,
                    )
                    r.start()
                    rdmas[(ring, sub)] = r
        if t < N_DEV - 1:
            with _scope("p_mxu"):
                c_cw = lax.rem(my - (t + 1) + N_DEV, N_DEV)
                c_ccw = lax.rem(my + t + 1, N_DEV)
                pa = pchunk(c_cw, 0, HALF)
                pb = pchunk(c_ccw, HALF, N_OUT)

        with _scope("p_waitacc"):
            for sub in range(2):
                lo, hi = sub * SUB, (sub + 1) * SUB
                for ring, comm, p in (("cw", comm_cw, "pa"), ("ccw", comm_ccw, "pb")):
                    rdmas[(ring, sub)].wait()
                    v = pa if p == "pa" else pb
                    if t < N_DEV - 2:
                        comm[r_slot, lo:hi, :] = comm[r_slot, lo:hi, :] + v[lo:hi, :]
                    elif t == N_DEV - 2:
                        comm[r_slot, lo:hi, :] = _silu(
                            comm[r_slot, lo:hi, :] + v[lo:hi, :])

        if pending_stores:
            with _scope("p_storewait"):
                for st in pending_stores:
                    st.wait()
            pending_stores = []
        if t < N_STEPS - (SLOTS - 1):
            pl.semaphore_signal(
                credit_cw, inc=1,
                device_id=(left,), device_id_type=pl.DeviceIdType.MESH,
            )
            pl.semaphore_signal(
                credit_ccw, inc=1,
                device_id=(right,), device_id_type=pl.DeviceIdType.MESH,
            )
        if t >= N_DEV - 2:
            c_a = lax.rem(my - (t - (N_DEV - 1)) + N_DEV, N_DEV)
            c_b = lax.rem(my + (t - (N_DEV - 1)) + N_DEV, N_DEV)
            st_a = pltpu.make_async_copy(
                comm_cw.at[r_slot],
                out_hbm.at[pl.ds(c_a * CHUNK, CHUNK), pl.ds(0, HALF)],
                out_sem_a)
            st_b = pltpu.make_async_copy(
                comm_ccw.at[r_slot],
                out_hbm.at[pl.ds(c_b * CHUNK, CHUNK), pl.ds(HALF, HALF)],
                out_sem_b)
            st_a.start()
            st_b.start()
            pending_stores = [st_a, st_b]

    for st in pending_stores:
        st.wait()


def kernel(x, w_mat):
    return pl.pallas_call(
        _body,
        out_shape=jax.ShapeDtypeStruct((M, N_OUT), jnp.float32),
        in_specs=[
            pl.BlockSpec(memory_space=pltpu.VMEM),
            pl.BlockSpec(memory_space=pltpu.VMEM),
        ],
        out_specs=pl.BlockSpec(memory_space=pltpu.MemorySpace.HBM),
        scratch_shapes=[
            pltpu.VMEM((SLOTS, CHUNK, HALF), jnp.float32),
            pltpu.VMEM((SLOTS, CHUNK, HALF), jnp.float32),
            pltpu.SemaphoreType.DMA((SLOTS, 2)),
            pltpu.SemaphoreType.DMA((SLOTS, 2)),
            pltpu.SemaphoreType.DMA((SLOTS, 2)),
            pltpu.SemaphoreType.DMA((SLOTS, 2)),
            pltpu.SemaphoreType.DMA,
            pltpu.SemaphoreType.DMA,
            pltpu.SemaphoreType.REGULAR,
            pltpu.SemaphoreType.REGULAR,
        ],
        compiler_params=pltpu.CompilerParams(collective_id=0),
    )(x, w_mat)


# device time: 353490 ns/iter; 1.1071x vs baseline; 1.0828x over previous
import contextlib
import os

import jax
import jax.numpy as jnp
from jax import lax
from jax.experimental import pallas as pl
from jax.experimental.pallas import tpu as pltpu

if os.environ.get("PROF_SCOPES"):
    def _scope(n):
        return jax.named_scope(n)
else:
    def _scope(n):
        return contextlib.nullcontext()

N_DEV = 8
M = 4096
K_SHARD = 512
N_OUT = 2048
HALF = N_OUT // 2
CHUNK = M // N_DEV
SUB = CHUNK // 2

N_STEPS = 2 * (N_DEV - 1)
SLOTS = 4


def _silu(v):
    return v * (1.0 / (1.0 + jnp.exp(-v)))


def _body(x_ref, w_ref, out_hbm, comm_cw, comm_ccw,
          send_cw, recv_cw, send_ccw, recv_ccw,
          out_sem_a, out_sem_b, credit_cw, credit_ccw):
    my = lax.axis_index("i")
    left = lax.rem(my + N_DEV - 1, N_DEV)
    right = lax.rem(my + 1, N_DEV)

    def pchunk(c, lo, hi):
        return jnp.dot(
            x_ref[pl.ds(c * CHUNK, CHUNK), :], w_ref[:, lo:hi],
            preferred_element_type=jnp.float32,
        )

    with _scope("p_seed"):
        comm_cw[0] = pchunk(my, 0, HALF)
        comm_ccw[0] = pchunk(my, HALF, N_OUT)

    barrier_sem = pltpu.get_barrier_semaphore()
    for nbr in (left, right):
        pl.semaphore_signal(
            barrier_sem, inc=1,
            device_id=(nbr,), device_id_type=pl.DeviceIdType.MESH,
        )
    pl.semaphore_wait(barrier_sem, 2)

    rings = (
        (comm_cw, send_cw, recv_cw, right),
        (comm_ccw, send_ccw, recv_ccw, left),
    )

    def mk_rdma(ring, t, sub):
        comm, ssem, rsem, dev = ring
        s_slot, r_slot = t % SLOTS, (t + 1) % SLOTS
        return pltpu.make_async_remote_copy(
            src_ref=comm.at[s_slot, pl.ds(sub * SUB, SUB), :],
            dst_ref=comm.at[r_slot, pl.ds(sub * SUB, SUB), :],
            send_sem=ssem.at[s_slot, sub],
            recv_sem=rsem.at[r_slot, sub],
            device_id=(dev,), device_id_type=pl.DeviceIdType.MESH,
        )

    send_pend = {0: []}
    with _scope("p_issue0"):
        for ring in rings:
            for sub in range(2):
                r = mk_rdma(ring, 0, sub)
                r.start()
                send_pend[0].append(r)

    pending_stores = []
    for t in range(N_STEPS):
        r_slot = (t + 1) % SLOTS
        if t < N_DEV - 1:
            with _scope("p_mxu"):
                c_cw = lax.rem(my - (t + 1) + N_DEV, N_DEV)
                c_ccw = lax.rem(my + t + 1, N_DEV)
                pa = pchunk(c_cw, 0, HALF)
                pb = pchunk(c_ccw, HALF, N_OUT)

        with _scope("p_waitacc"):
            for sub in range(2):
                lo, hi = sub * SUB, (sub + 1) * SUB
                for ri, ring in enumerate(rings):
                    comm = ring[0]
                    mk_rdma(ring, t, sub).wait_recv()
                    if t < N_DEV - 1:
                        v = pa if ri == 0 else pb
                        if t < N_DEV - 2:
                            comm[r_slot, lo:hi, :] = (
                                comm[r_slot, lo:hi, :] + v[lo:hi, :])
                        else:
                            comm[r_slot, lo:hi, :] = _silu(
                                comm[r_slot, lo:hi, :] + v[lo:hi, :])
                if t + 1 < N_STEPS:
                    if sub == 0 and t + 1 >= SLOTS - 1:
                        with _scope("p_credit"):
                            pl.semaphore_wait(credit_cw, 1)
                            pl.semaphore_wait(credit_ccw, 1)
                    send_pend.setdefault(t + 1, [])
                    for ring in rings:
                        r = mk_rdma(ring, t + 1, sub)
                        r.start()
                        send_pend[t + 1].append(r)

        with _scope("p_sendwait"):
            for r in send_pend.pop(t, []):
                r.wait_send()
        if pending_stores:
            with _scope("p_storewait"):
                for st in pending_stores:
                    st.wait()
            pending_stores = []
        if t < N_STEPS - (SLOTS - 1):
            pl.semaphore_signal(
                credit_cw, inc=1,
                device_id=(left,), device_id_type=pl.DeviceIdType.MESH,
            )
            pl.semaphore_signal(
                credit_ccw, inc=1,
                device_id=(right,), device_id_type=pl.DeviceIdType.MESH,
            )
        if t >= N_DEV - 2:
            c_a = lax.rem(my - (t - (N_DEV - 1)) + N_DEV, N_DEV)
            c_b = lax.rem(my + (t - (N_DEV - 1)) + N_DEV, N_DEV)
            st_a = pltpu.make_async_copy(
                comm_cw.at[r_slot],
                out_hbm.at[pl.ds(c_a * CHUNK, CHUNK), pl.ds(0, HALF)],
                out_sem_a)
            st_b = pltpu.make_async_copy(
                comm_ccw.at[r_slot],
                out_hbm.at[pl.ds(c_b * CHUNK, CHUNK), pl.ds(HALF, HALF)],
                out_sem_b)
            st_a.start()
            st_b.start()
            pending_stores = [st_a, st_b]

    for st in pending_stores:
        st.wait()


def kernel(x, w_mat):
    return pl.pallas_call(
        _body,
        out_shape=jax.ShapeDtypeStruct((M, N_OUT), jnp.float32),
        in_specs=[
            pl.BlockSpec(memory_space=pltpu.VMEM),
            pl.BlockSpec(memory_space=pltpu.VMEM),
        ],
        out_specs=pl.BlockSpec(memory_space=pltpu.MemorySpace.HBM),
        scratch_shapes=[
            pltpu.VMEM((SLOTS, CHUNK, HALF), jnp.float32),
            pltpu.VMEM((SLOTS, CHUNK, HALF), jnp.float32),
            pltpu.SemaphoreType.DMA((SLOTS, 2)),
            pltpu.SemaphoreType.DMA((SLOTS, 2)),
            pltpu.SemaphoreType.DMA((SLOTS, 2)),
            pltpu.SemaphoreType.DMA((SLOTS, 2)),
            pltpu.SemaphoreType.DMA,
            pltpu.SemaphoreType.DMA,
            pltpu.SemaphoreType.REGULAR,
            pltpu.SemaphoreType.REGULAR,
        ],
        compiler_params=pltpu.CompilerParams(collective_id=0),
    )(x, w_mat)


# device time: 352552 ns/iter; 1.1101x vs baseline; 1.0027x over previous
import contextlib
import os

import jax
import jax.numpy as jnp
from jax import lax
from jax.experimental import pallas as pl
from jax.experimental.pallas import tpu as pltpu

if os.environ.get("PROF_SCOPES"):
    def _scope(n):
        return jax.named_scope(n)
else:
    def _scope(n):
        return contextlib.nullcontext()

N_DEV = 8
M = 4096
K_SHARD = 512
N_OUT = 2048
HALF = N_OUT // 2
CHUNK = M // N_DEV
SUBS = 4
SUB = CHUNK // SUBS

N_STEPS = 2 * (N_DEV - 1)
SLOTS = 4


def _silu(v):
    return v * (1.0 / (1.0 + jnp.exp(-v)))


def _body(x_ref, w_ref, out_hbm, comm_cw, comm_ccw,
          send_cw, recv_cw, send_ccw, recv_ccw,
          out_sem_a, out_sem_b, credit_cw, credit_ccw):
    my = lax.axis_index("i")
    left = lax.rem(my + N_DEV - 1, N_DEV)
    right = lax.rem(my + 1, N_DEV)

    def pchunk(c, lo, hi):
        return jnp.dot(
            x_ref[pl.ds(c * CHUNK, CHUNK), :], w_ref[:, lo:hi],
            preferred_element_type=jnp.float32,
        )

    barrier_sem = pltpu.get_barrier_semaphore()
    for nbr in (left, right):
        pl.semaphore_signal(
            barrier_sem, inc=1,
            device_id=(nbr,), device_id_type=pl.DeviceIdType.MESH,
        )

    rings = (
        (comm_cw, send_cw, recv_cw, right),
        (comm_ccw, send_ccw, recv_ccw, left),
    )

    def mk_rdma(ring, t, sub):
        comm, ssem, rsem, dev = ring
        s_slot, r_slot = t % SLOTS, (t + 1) % SLOTS
        return pltpu.make_async_remote_copy(
            src_ref=comm.at[s_slot, pl.ds(sub * SUB, SUB), :],
            dst_ref=comm.at[r_slot, pl.ds(sub * SUB, SUB), :],
            send_sem=ssem.at[s_slot, sub],
            recv_sem=rsem.at[r_slot, sub],
            device_id=(dev,), device_id_type=pl.DeviceIdType.MESH,
        )

    send_pend = {0: []}
    with _scope("p_seed"):
        for sub in range(SUBS):
            lo = sub * SUB
            row0 = my * CHUNK + lo
            comm_cw[0, lo:lo + SUB, :] = jnp.dot(
                x_ref[pl.ds(row0, SUB), :], w_ref[:, 0:HALF],
                preferred_element_type=jnp.float32)
            comm_ccw[0, lo:lo + SUB, :] = jnp.dot(
                x_ref[pl.ds(row0, SUB), :], w_ref[:, HALF:N_OUT],
                preferred_element_type=jnp.float32)
            if sub == 0:
                pl.semaphore_wait(barrier_sem, 2)
            for ring in rings:
                r = mk_rdma(ring, 0, sub)
                r.start()
                send_pend[0].append(r)

    pending_stores = []
    for t in range(N_STEPS):
        r_slot = (t + 1) % SLOTS
        if t < N_DEV - 1:
            with _scope("p_mxu"):
                c_cw = lax.rem(my - (t + 1) + N_DEV, N_DEV)
                c_ccw = lax.rem(my + t + 1, N_DEV)
                pa = pchunk(c_cw, 0, HALF)
                pb = pchunk(c_ccw, HALF, N_OUT)

        with _scope("p_waitacc"):
            for sub in range(SUBS):
                lo, hi = sub * SUB, (sub + 1) * SUB
                for ri, ring in enumerate(rings):
                    comm = ring[0]
                    mk_rdma(ring, t, sub).wait_recv()
                    if t < N_DEV - 1:
                        v = pa if ri == 0 else pb
                        if t < N_DEV - 2:
                            comm[r_slot, lo:hi, :] = (
                                comm[r_slot, lo:hi, :] + v[lo:hi, :])
                        else:
                            comm[r_slot, lo:hi, :] = _silu(
                                comm[r_slot, lo:hi, :] + v[lo:hi, :])
                if t + 1 < N_STEPS:
                    if sub == 0 and t + 1 >= SLOTS - 1:
                        with _scope("p_credit"):
                            pl.semaphore_wait(credit_cw, 1)
                            pl.semaphore_wait(credit_ccw, 1)
                    send_pend.setdefault(t + 1, [])
                    for ring in rings:
                        r = mk_rdma(ring, t + 1, sub)
                        r.start()
                        send_pend[t + 1].append(r)

        with _scope("p_sendwait"):
            for r in send_pend.pop(t, []):
                r.wait_send()
        if pending_stores:
            with _scope("p_storewait"):
                for st in pending_stores:
                    st.wait()
            pending_stores = []
        if t < N_STEPS - (SLOTS - 1):
            pl.semaphore_signal(
                credit_cw, inc=1,
                device_id=(left,), device_id_type=pl.DeviceIdType.MESH,
            )
            pl.semaphore_signal(
                credit_ccw, inc=1,
                device_id=(right,), device_id_type=pl.DeviceIdType.MESH,
            )
        if t >= N_DEV - 2:
            c_a = lax.rem(my - (t - (N_DEV - 1)) + N_DEV, N_DEV)
            c_b = lax.rem(my + (t - (N_DEV - 1)) + N_DEV, N_DEV)
            st_a = pltpu.make_async_copy(
                comm_cw.at[r_slot],
                out_hbm.at[pl.ds(c_a * CHUNK, CHUNK), pl.ds(0, HALF)],
                out_sem_a)
            st_b = pltpu.make_async_copy(
                comm_ccw.at[r_slot],
                out_hbm.at[pl.ds(c_b * CHUNK, CHUNK), pl.ds(HALF, HALF)],
                out_sem_b)
            st_a.start()
            st_b.start()
            pending_stores = [st_a, st_b]

    for st in pending_stores:
        st.wait()


def kernel(x, w_mat):
    return pl.pallas_call(
        _body,
        out_shape=jax.ShapeDtypeStruct((M, N_OUT), jnp.float32),
        in_specs=[
            pl.BlockSpec(memory_space=pltpu.VMEM),
            pl.BlockSpec(memory_space=pltpu.VMEM),
        ],
        out_specs=pl.BlockSpec(memory_space=pltpu.MemorySpace.HBM),
        scratch_shapes=[
            pltpu.VMEM((SLOTS, CHUNK, HALF), jnp.float32),
            pltpu.VMEM((SLOTS, CHUNK, HALF), jnp.float32),
            pltpu.SemaphoreType.DMA((SLOTS, SUBS)),
            pltpu.SemaphoreType.DMA((SLOTS, SUBS)),
            pltpu.SemaphoreType.DMA((SLOTS, SUBS)),
            pltpu.SemaphoreType.DMA((SLOTS, SUBS)),
            pltpu.SemaphoreType.DMA,
            pltpu.SemaphoreType.DMA,
            pltpu.SemaphoreType.REGULAR,
            pltpu.SemaphoreType.REGULAR,
        ],
        compiler_params=pltpu.CompilerParams(collective_id=0),
    )(x, w_mat)


# device time: 352459 ns/iter; 1.1104x vs baseline; 1.0003x over previous
import contextlib
import os

import jax
import jax.numpy as jnp
from jax import lax
from jax.experimental import pallas as pl
from jax.experimental.pallas import tpu as pltpu

if os.environ.get("PROF_SCOPES"):
    def _scope(n):
        return jax.named_scope(n)
else:
    def _scope(n):
        return contextlib.nullcontext()

N_DEV = 8
M = 4096
K_SHARD = 512
N_OUT = 2048
HALF = N_OUT // 2
CHUNK = M // N_DEV
SUBS = 4
SUB = CHUNK // SUBS

N_STEPS = 2 * (N_DEV - 1)
SLOTS = 4


def _silu(v):
    return v * (1.0 / (1.0 + jnp.exp(-v)))


def _body(x_ref, w_ref, out_hbm, comm_cw, comm_ccw,
          send_cw, recv_cw, send_ccw, recv_ccw,
          out_sem_a, out_sem_b, credit_cw, credit_ccw):
    my = lax.axis_index("i")
    left = lax.rem(my + N_DEV - 1, N_DEV)
    right = lax.rem(my + 1, N_DEV)

    def pchunk(c, lo, hi):
        return jnp.dot(
            x_ref[pl.ds(c * CHUNK, CHUNK), :], w_ref[:, lo:hi],
            preferred_element_type=jnp.float32,
        )

    barrier_sem = pltpu.get_barrier_semaphore()
    for nbr in (left, right):
        pl.semaphore_signal(
            barrier_sem, inc=1,
            device_id=(nbr,), device_id_type=pl.DeviceIdType.MESH,
        )

    rings = (
        (comm_cw, send_cw, recv_cw, right),
        (comm_ccw, send_ccw, recv_ccw, left),
    )

    def mk_rdma(ring, t, sub):
        comm, ssem, rsem, dev = ring
        s_slot, r_slot = t % SLOTS, (t + 1) % SLOTS
        return pltpu.make_async_remote_copy(
            src_ref=comm.at[s_slot, pl.ds(sub * SUB, SUB), :],
            dst_ref=comm.at[r_slot, pl.ds(sub * SUB, SUB), :],
            send_sem=ssem.at[s_slot, sub],
            recv_sem=rsem.at[r_slot, sub],
            device_id=(dev,), device_id_type=pl.DeviceIdType.MESH,
        )

    send_pend = {0: []}
    with _scope("p_seed"):
        for sub in range(SUBS):
            lo = sub * SUB
            row0 = my * CHUNK + lo
            comm_cw[0, lo:lo + SUB, :] = jnp.dot(
                x_ref[pl.ds(row0, SUB), :], w_ref[:, 0:HALF],
                preferred_element_type=jnp.float32)
            comm_ccw[0, lo:lo + SUB, :] = jnp.dot(
                x_ref[pl.ds(row0, SUB), :], w_ref[:, HALF:N_OUT],
                preferred_element_type=jnp.float32)
            if sub == 0:
                pl.semaphore_wait(barrier_sem, 2)
            for ring in rings:
                r = mk_rdma(ring, 0, sub)
                r.start()
                send_pend[0].append(r)

    pending_stores = []
    for t in range(N_STEPS):
        r_slot = (t + 1) % SLOTS
        if t < N_DEV - 1:
            with _scope("p_mxu"):
                c_cw = lax.rem(my - (t + 1) + N_DEV, N_DEV)
                c_ccw = lax.rem(my + t + 1, N_DEV)
                pa = pchunk(c_cw, 0, HALF)
                pb = pchunk(c_ccw, HALF, N_OUT)

        with _scope("p_waitacc"):
            for sub in range(SUBS):
                lo, hi = sub * SUB, (sub + 1) * SUB
                for ri, ring in enumerate(rings):
                    comm = ring[0]
                    mk_rdma(ring, t, sub).wait_recv()
                    if t < N_DEV - 1:
                        v = pa if ri == 0 else pb
                        if t < N_DEV - 2:
                            comm[r_slot, lo:hi, :] = (
                                comm[r_slot, lo:hi, :] + v[lo:hi, :])
                        else:
                            comm[r_slot, lo:hi, :] = _silu(
                                comm[r_slot, lo:hi, :] + v[lo:hi, :])
                if t + 1 < N_STEPS:
                    if sub == 0 and t + 1 >= SLOTS - 1:
                        with _scope("p_credit"):
                            pl.semaphore_wait(credit_cw, 1)
                            pl.semaphore_wait(credit_ccw, 1)
                    send_pend.setdefault(t + 1, [])
                    for ring in rings:
                        r = mk_rdma(ring, t + 1, sub)
                        r.start()
                        send_pend[t + 1].append(r)

        with _scope("p_sendwait"):
            for r in send_pend.pop(t, []):
                r.wait_send()
        if pending_stores:
            with _scope("p_storewait"):
                for st in pending_stores:
                    st.wait()
            pending_stores = []
        if t < N_STEPS - (SLOTS - 1):
            pl.semaphore_signal(
                credit_cw, inc=1,
                device_id=(left,), device_id_type=pl.DeviceIdType.MESH,
            )
            pl.semaphore_signal(
                credit_ccw, inc=1,
                device_id=(right,), device_id_type=pl.DeviceIdType.MESH,
            )
        if t >= N_DEV - 2:
            c_a = lax.rem(my - (t - (N_DEV - 1)) + N_DEV, N_DEV)
            c_b = lax.rem(my + (t - (N_DEV - 1)) + N_DEV, N_DEV)
            st_a = pltpu.make_async_copy(
                comm_cw.at[r_slot],
                out_hbm.at[pl.ds(c_a * CHUNK, CHUNK), pl.ds(0, HALF)],
                out_sem_a)
            st_b = pltpu.make_async_copy(
                comm_ccw.at[r_slot],
                out_hbm.at[pl.ds(c_b * CHUNK, CHUNK), pl.ds(HALF, HALF)],
                out_sem_b)
            st_a.start()
            st_b.start()
            pending_stores = [st_a, st_b]

    for st in pending_stores:
        st.wait()


def kernel(x, w_mat):
    return pl.pallas_call(
        _body,
        out_shape=jax.ShapeDtypeStruct((M, N_OUT), jnp.float32),
        in_specs=[
            pl.BlockSpec(memory_space=pltpu.VMEM),
            pl.BlockSpec(memory_space=pltpu.VMEM),
        ],
        out_specs=pl.BlockSpec(memory_space=pl.MemorySpace.ANY),
        scratch_shapes=[
            pltpu.VMEM((SLOTS, CHUNK, HALF), jnp.float32),
            pltpu.VMEM((SLOTS, CHUNK, HALF), jnp.float32),
            pltpu.SemaphoreType.DMA((SLOTS, SUBS)),
            pltpu.SemaphoreType.DMA((SLOTS, SUBS)),
            pltpu.SemaphoreType.DMA((SLOTS, SUBS)),
            pltpu.SemaphoreType.DMA((SLOTS, SUBS)),
            pltpu.SemaphoreType.DMA,
            pltpu.SemaphoreType.DMA,
            pltpu.SemaphoreType.REGULAR,
            pltpu.SemaphoreType.REGULAR,
        ],
        compiler_params=pltpu.CompilerParams(collective_id=0),
    )(x, w_mat)


# device time: 283231 ns/iter; 1.3818x vs baseline; 1.2444x over previous
import contextlib
import os

import jax
import jax.numpy as jnp
from jax import lax
from jax.experimental import pallas as pl
from jax.experimental.pallas import tpu as pltpu

M = 4096
K_SHARD = 512
N_OUT = 2048

SNAKE = [(0, 0, 0), (1, 0, 0), (1, 1, 0), (0, 1, 0),
         (0, 0, 1), (1, 0, 1), (1, 1, 1), (0, 1, 1)]
POS = {c: i for i, c in enumerate(SNAKE)}
BIT = [[c[d] for c in SNAKE] for d in range(3)]
NBR = [[POS[tuple(c[i] ^ (i == d) for i in range(3))] for c in SNAKE]
       for d in range(3)]

PARTS = [
    (0, 768, (0, 1, 2)),
    (768, 1408, (1, 2, 0)),
    (1408, 2048, (2, 0, 1)),
]

if os.environ.get("PROF_SCOPES"):
    def _scope(n):
        return jax.named_scope(n)
else:
    def _scope(n):
        return contextlib.nullcontext()


def _silu(v):
    return v * (1.0 / (1.0 + jnp.exp(-v)))


def _tlu(tbl, idx):
    out = jnp.int32(0)
    for k in range(8):
        out = out + jnp.where(idx == k, jnp.int32(tbl[k]), jnp.int32(0))
    return out


def _body(x_ref, w_ref, out_hbm,
          acc0, acc1, acc2, recv0, recv1, recv2, snd0, snd1, snd2,
          send_sems, recv_sems, out_sems):
    my = lax.axis_index("i")
    bits = [_tlu(BIT[d], my) for d in range(3)]
    nbrs = [_tlu(NBR[d], my) for d in range(3)]

    accs = (acc0, acc1, acc2)
    recvs = (recv0, recv1, recv2)
    snds = (snd0, snd1, snd2)

    barrier_sem = pltpu.get_barrier_semaphore()
    for d in range(3):
        pl.semaphore_signal(
            barrier_sem, inc=1,
            device_id=(nbrs[d],), device_id_type=pl.DeviceIdType.MESH,
        )

    def pdot(row_start, nrows, lo, hi):
        return jnp.dot(
            x_ref[pl.ds(row_start, nrows), :], w_ref[:, lo:hi],
            preferred_element_type=jnp.float32,
        )

    def exch(p, e, src, dst, dev):
        return pltpu.make_async_remote_copy(
            src_ref=src, dst_ref=dst,
            send_sem=send_sems.at[p, e], recv_sem=recv_sems.at[p, e],
            device_id=(dev,), device_id_type=pl.DeviceIdType.MESH,
        )

    NP = len(PARTS)
    half = [None] * NP
    off2 = [None] * NP
    off3 = [None] * NP
    ex = {}

    with _scope("rs1_sub0"):
        for p, (lo, hi, dims) in enumerate(PARTS):
            b1 = bits[dims[0]]
            peer_base = (1 - b1) * 2048
            snds[p][...] = pdot(peer_base, 1024, lo, hi)
            if p == 0:
                pl.semaphore_wait(barrier_sem, 3)
            r = exch(p, 0, snds[p], recvs[p].at[0:1024, :], nbrs[dims[0]])
            r.start()
            ex[(p, 0)] = r
    with _scope("rs1_keepdot"):
        for p, (lo, hi, dims) in enumerate(PARTS):
            b1 = bits[dims[0]]
            half[p] = b1 * 2048
            accs[p][...] = pdot(half[p], 2048, lo, hi)
    with _scope("rs1_sub1"):
        for p, (lo, hi, dims) in enumerate(PARTS):
            b1 = bits[dims[0]]
            peer_base = (1 - b1) * 2048
            ex[(p, 0)].wait_send()
            snds[p][...] = pdot(peer_base + 1024, 1024, lo, hi)
            r = exch(p, 1, snds[p], recvs[p].at[1024:2048, :], nbrs[dims[0]])
            r.start()
            ex[(p, 1)] = r

    ORDER = (1, 2, 0)

    with _scope("rs2"):
        for p in ORDER:
            lo, hi, dims = PARTS[p]
            b2 = bits[dims[1]]
            ex[(p, 0)].wait_recv()
            ex[(p, 1)].wait_recv()
            ex[(p, 1)].wait_send()
            accs[p][...] = accs[p][...] + recvs[p][...]
            off2[p] = b2 * 1024
            send_off = (1 - b2) * 1024
            r = exch(p, 2, accs[p].at[pl.ds(send_off, 1024), :],
                     recvs[p].at[0:1024, :], nbrs[dims[1]])
            r.start()
            ex[(p, 2)] = r

    with _scope("rs3"):
        for p in ORDER:
            lo, hi, dims = PARTS[p]
            b3 = bits[dims[2]]
            ex[(p, 2)].wait_recv()
            ex[(p, 2)].wait_send()
            accs[p][pl.ds(off2[p], 1024), :] = (
                accs[p][pl.ds(off2[p], 1024), :] + recvs[p][0:1024, :])
            off3[p] = off2[p] + b3 * 512
            send_off = off2[p] + (1 - b3) * 512
            r = exch(p, 3, accs[p].at[pl.ds(send_off, 512), :],
                     recvs[p].at[0:512, :], nbrs[dims[2]])
            r.start()
            ex[(p, 3)] = r

    stores = []

    with _scope("ag3"):
        for p in ORDER:
            lo, hi, dims = PARTS[p]
            ex[(p, 3)].wait_recv()
            ex[(p, 3)].wait_send()
            v = _silu(accs[p][pl.ds(off3[p], 512), :] + recvs[p][0:512, :])
            accs[p][pl.ds(off3[p], 512), :] = v
            r = exch(p, 4, accs[p].at[pl.ds(off3[p], 512), :],
                     accs[p].at[pl.ds(off3[p], 512), :], nbrs[dims[2]])
            r.start()
            ex[(p, 4)] = r
            st = pltpu.make_async_copy(
                accs[p].at[pl.ds(off3[p], 512), :],
                out_hbm.at[pl.ds(half[p] + off3[p], 512), pl.ds(lo, hi - lo)],
                out_sems.at[p, 0])
            st.start()
            stores.append(st)

    with _scope("ag2"):
        for p in ORDER:
            lo, hi, dims = PARTS[p]
            b3 = bits[dims[2]]
            sib3 = off2[p] + (1 - b3) * 512
            ex[(p, 4)].wait_recv()
            ex[(p, 4)].wait_send()
            st = pltpu.make_async_copy(
                accs[p].at[pl.ds(sib3, 512), :],
                out_hbm.at[pl.ds(half[p] + sib3, 512), pl.ds(lo, hi - lo)],
                out_sems.at[p, 1])
            st.start()
            stores.append(st)
            r = exch(p, 5, accs[p].at[pl.ds(off2[p], 1024), :],
                     accs[p].at[pl.ds(off2[p], 1024), :], nbrs[dims[1]])
            r.start()
            ex[(p, 5)] = r

    with _scope("ag1"):
        for p in ORDER:
            lo, hi, dims = PARTS[p]
            b2 = bits[dims[1]]
            sib2 = (1 - b2) * 1024
            ex[(p, 5)].wait_recv()
            ex[(p, 5)].wait_send()
            st = pltpu.make_async_copy(
                accs[p].at[pl.ds(sib2, 1024), :],
                out_hbm.at[pl.ds(half[p] + sib2, 1024), pl.ds(lo, hi - lo)],
                out_sems.at[p, 2])
            st.start()
            stores.append(st)
            r = exch(p, 6, accs[p], recvs[p], nbrs[dims[0]])
            r.start()
            ex[(p, 6)] = r

    with _scope("tail"):
        for p in ORDER:
            lo, hi, dims = PARTS[p]
            b1 = bits[dims[0]]
            other = (1 - b1) * 2048
            ex[(p, 6)].wait_recv()
            ex[(p, 6)].wait_send()
            st = pltpu.make_async_copy(
                recvs[p],
                out_hbm.at[pl.ds(other, 2048), pl.ds(lo, hi - lo)],
                out_sems.at[p, 3])
            st.start()
            stores.append(st)
        for st in stores:
            st.wait()


def kernel(x, w_mat):
    scratch = []
    for _, (lo, hi, _dims) in [(0, p) for p in PARTS]:
        scratch.append(pltpu.VMEM((2048, hi - lo), jnp.float32))
    for _, (lo, hi, _dims) in [(0, p) for p in PARTS]:
        scratch.append(pltpu.VMEM((2048, hi - lo), jnp.float32))
    for _, (lo, hi, _dims) in [(0, p) for p in PARTS]:
        scratch.append(pltpu.VMEM((1024, hi - lo), jnp.float32))
    scratch += [
        pltpu.SemaphoreType.DMA((3, 7)),
        pltpu.SemaphoreType.DMA((3, 7)),
        pltpu.SemaphoreType.DMA((3, 4)),
    ]
    return pl.pallas_call(
        _body,
        out_shape=jax.ShapeDtypeStruct((M, N_OUT), jnp.float32),
        in_specs=[
            pl.BlockSpec(memory_space=pltpu.VMEM),
            pl.BlockSpec(memory_space=pltpu.VMEM),
        ],
        out_specs=pl.BlockSpec(memory_space=pl.MemorySpace.ANY),
        scratch_shapes=scratch,
        compiler_params=pltpu.CompilerParams(
            collective_id=0, vmem_limit_bytes=60 * 1024 * 1024),
    )(x, w_mat)


# device time: 283075 ns/iter; 1.3826x vs baseline; 1.0006x over previous
import contextlib
import os

import jax
import jax.numpy as jnp
from jax import lax
from jax.experimental import pallas as pl
from jax.experimental.pallas import tpu as pltpu

M = 4096
K_SHARD = 512
N_OUT = 2048

SNAKE = [(0, 0, 0), (1, 0, 0), (1, 1, 0), (0, 1, 0),
         (0, 0, 1), (1, 0, 1), (1, 1, 1), (0, 1, 1)]
POS = {c: i for i, c in enumerate(SNAKE)}
BIT = [[c[d] for c in SNAKE] for d in range(3)]
NBR = [[POS[tuple(c[i] ^ (i == d) for i in range(3))] for c in SNAKE]
       for d in range(3)]

PARTS = [
    (0, 768, (0, 1, 2)),
    (768, 1408, (1, 2, 0)),
    (1408, 2048, (2, 0, 1)),
]

if os.environ.get("PROF_SCOPES"):
    def _scope(n):
        return jax.named_scope(n)
else:
    def _scope(n):
        return contextlib.nullcontext()


def _silu(v):
    return v * (1.0 / (1.0 + jnp.exp(-v)))


def _tlu(tbl, idx):
    out = jnp.int32(0)
    for k in range(8):
        out = out + jnp.where(idx == k, jnp.int32(tbl[k]), jnp.int32(0))
    return out


def _body(x_ref, w_ref, out_hbm,
          acc0, acc1, acc2, recv0, recv1, recv2, snd0, snd1, snd2,
          send_sems, recv_sems, out_sems):
    my = lax.axis_index("i")
    bits = [_tlu(BIT[d], my) for d in range(3)]
    nbrs = [_tlu(NBR[d], my) for d in range(3)]

    accs = (acc0, acc1, acc2)
    recvs = (recv0, recv1, recv2)
    snds = (snd0, snd1, snd2)

    barrier_sem = pltpu.get_barrier_semaphore()
    for d in range(3):
        pl.semaphore_signal(
            barrier_sem, inc=1,
            device_id=(nbrs[d],), device_id_type=pl.DeviceIdType.MESH,
        )

    def pdot(row_start, nrows, lo, hi):
        return jnp.dot(
            x_ref[pl.ds(row_start, nrows), :], w_ref[:, lo:hi],
            preferred_element_type=jnp.float32,
        )

    def exch(p, e, src, dst, dev):
        return pltpu.make_async_remote_copy(
            src_ref=src, dst_ref=dst,
            send_sem=send_sems.at[p, e], recv_sem=recv_sems.at[p, e],
            device_id=(dev,), device_id_type=pl.DeviceIdType.MESH,
        )

    NP = len(PARTS)
    half = [None] * NP
    off2 = [None] * NP
    off3 = [None] * NP
    ex = {}

    with _scope("rs1_sub0"):
        for p, (lo, hi, dims) in enumerate(PARTS):
            b1 = bits[dims[0]]
            peer_base = (1 - b1) * 2048
            snds[p][...] = pdot(peer_base, 1024, lo, hi)
            if p == 0:
                pl.semaphore_wait(barrier_sem, 3)
            r = exch(p, 0, snds[p], recvs[p].at[0:1024, :], nbrs[dims[0]])
            r.start()
            ex[(p, 0)] = r
    with _scope("rs1_keepdot"):
        for p, (lo, hi, dims) in enumerate(PARTS):
            b1 = bits[dims[0]]
            half[p] = b1 * 2048
            accs[p][...] = pdot(half[p], 2048, lo, hi)
    with _scope("rs1_sub1"):
        for p, (lo, hi, dims) in enumerate(PARTS):
            b1 = bits[dims[0]]
            peer_base = (1 - b1) * 2048
            ex[(p, 0)].wait_send()
            snds[p][...] = pdot(peer_base + 1024, 1024, lo, hi)
            r = exch(p, 1, snds[p], recvs[p].at[1024:2048, :], nbrs[dims[0]])
            r.start()
            ex[(p, 1)] = r

    ORDER = (1, 2, 0)

    with _scope("rs2"):
        for p in ORDER:
            lo, hi, dims = PARTS[p]
            b2 = bits[dims[1]]
            ex[(p, 0)].wait_recv()
            accs[p][0:1024, :] = accs[p][0:1024, :] + recvs[p][0:1024, :]
            ex[(p, 1)].wait_recv()
            ex[(p, 1)].wait_send()
            accs[p][1024:2048, :] = (
                accs[p][1024:2048, :] + recvs[p][1024:2048, :])
            off2[p] = b2 * 1024
            send_off = (1 - b2) * 1024
            r = exch(p, 2, accs[p].at[pl.ds(send_off, 1024), :],
                     recvs[p].at[0:1024, :], nbrs[dims[1]])
            r.start()
            ex[(p, 2)] = r

    with _scope("rs3"):
        for p in ORDER:
            lo, hi, dims = PARTS[p]
            b3 = bits[dims[2]]
            ex[(p, 2)].wait_recv()
            ex[(p, 2)].wait_send()
            accs[p][pl.ds(off2[p], 1024), :] = (
                accs[p][pl.ds(off2[p], 1024), :] + recvs[p][0:1024, :])
            off3[p] = off2[p] + b3 * 512
            send_off = off2[p] + (1 - b3) * 512
            r = exch(p, 3, accs[p].at[pl.ds(send_off, 512), :],
                     recvs[p].at[0:512, :], nbrs[dims[2]])
            r.start()
            ex[(p, 3)] = r

    stores = []

    with _scope("ag3"):
        for p in ORDER:
            lo, hi, dims = PARTS[p]
            ex[(p, 3)].wait_recv()
            ex[(p, 3)].wait_send()
            v = _silu(accs[p][pl.ds(off3[p], 512), :] + recvs[p][0:512, :])
            accs[p][pl.ds(off3[p], 512), :] = v
            r = exch(p, 4, accs[p].at[pl.ds(off3[p], 512), :],
                     accs[p].at[pl.ds(off3[p], 512), :], nbrs[dims[2]])
            r.start()
            ex[(p, 4)] = r
            st = pltpu.make_async_copy(
                accs[p].at[pl.ds(off3[p], 512), :],
                out_hbm.at[pl.ds(half[p] + off3[p], 512), pl.ds(lo, hi - lo)],
                out_sems.at[p, 0])
            st.start()
            stores.append(st)

    with _scope("ag2"):
        for p in ORDER:
            lo, hi, dims = PARTS[p]
            b3 = bits[dims[2]]
            sib3 = off2[p] + (1 - b3) * 512
            ex[(p, 4)].wait_recv()
            ex[(p, 4)].wait_send()
            st = pltpu.make_async_copy(
                accs[p].at[pl.ds(sib3, 512), :],
                out_hbm.at[pl.ds(half[p] + sib3, 512), pl.ds(lo, hi - lo)],
                out_sems.at[p, 1])
            st.start()
            stores.append(st)
            r = exch(p, 5, accs[p].at[pl.ds(off2[p], 1024), :],
                     accs[p].at[pl.ds(off2[p], 1024), :], nbrs[dims[1]])
            r.start()
            ex[(p, 5)] = r

    with _scope("ag1"):
        for p in ORDER:
            lo, hi, dims = PARTS[p]
            b2 = bits[dims[1]]
            sib2 = (1 - b2) * 1024
            ex[(p, 5)].wait_recv()
            ex[(p, 5)].wait_send()
            st = pltpu.make_async_copy(
                accs[p].at[pl.ds(sib2, 1024), :],
                out_hbm.at[pl.ds(half[p] + sib2, 1024), pl.ds(lo, hi - lo)],
                out_sems.at[p, 2])
            st.start()
            stores.append(st)
            r = exch(p, 6, accs[p], recvs[p], nbrs[dims[0]])
            r.start()
            ex[(p, 6)] = r

    with _scope("tail"):
        for p in ORDER:
            lo, hi, dims = PARTS[p]
            b1 = bits[dims[0]]
            other = (1 - b1) * 2048
            ex[(p, 6)].wait_recv()
            ex[(p, 6)].wait_send()
            st = pltpu.make_async_copy(
                recvs[p],
                out_hbm.at[pl.ds(other, 2048), pl.ds(lo, hi - lo)],
                out_sems.at[p, 3])
            st.start()
            stores.append(st)
        for st in stores:
            st.wait()


def kernel(x, w_mat):
    scratch = []
    for _, (lo, hi, _dims) in [(0, p) for p in PARTS]:
        scratch.append(pltpu.VMEM((2048, hi - lo), jnp.float32))
    for _, (lo, hi, _dims) in [(0, p) for p in PARTS]:
        scratch.append(pltpu.VMEM((2048, hi - lo), jnp.float32))
    for _, (lo, hi, _dims) in [(0, p) for p in PARTS]:
        scratch.append(pltpu.VMEM((1024, hi - lo), jnp.float32))
    scratch += [
        pltpu.SemaphoreType.DMA((3, 7)),
        pltpu.SemaphoreType.DMA((3, 7)),
        pltpu.SemaphoreType.DMA((3, 4)),
    ]
    return pl.pallas_call(
        _body,
        out_shape=jax.ShapeDtypeStruct((M, N_OUT), jnp.float32),
        in_specs=[
            pl.BlockSpec(memory_space=pltpu.VMEM),
            pl.BlockSpec(memory_space=pltpu.VMEM),
        ],
        out_specs=pl.BlockSpec(memory_space=pl.MemorySpace.ANY),
        scratch_shapes=scratch,
        compiler_params=pltpu.CompilerParams(
            collective_id=0, vmem_limit_bytes=60 * 1024 * 1024),
    )(x, w_mat)
